# baseline (speedup 1.0000x reference)
import functools

import jax
import jax.numpy as jnp
from jax import lax
from jax.experimental import pallas as pl
from jax.experimental.pallas import tpu as pltpu
from jax.experimental.pallas import tpu_sc as plsc

N = 100000
D = 128
G = 256
NC = 2
NS = 16
NW = NC * NS
L = 16

GRP = 16
NG = N // GRP
GW = 208
NGA = NG // 8 * 8
TAIL = NG - NGA
TCB = 10000


def _tc_group_max(x):
    def body(x_ref, out_ref):
        v = x_ref[...].reshape(TCB // GRP, GRP, D)
        out_ref[...] = jnp.max(v, axis=1)[None]

    out = pl.pallas_call(
        body,
        grid=(N // TCB,),
        in_specs=[pl.BlockSpec((TCB, D), lambda i: (i, 0))],
        out_specs=pl.BlockSpec((1, TCB // GRP, D), lambda i: (i, 0, 0)),
        out_shape=jax.ShapeDtypeStruct((N // TCB, TCB // GRP, D),
                                       jnp.float32),
    )(x)
    return out.reshape(NG, D)


def _sc_segment_max_partial(xg, x, batch):
    mesh = plsc.VectorSubcoreMesh(core_axis_name="c", subcore_axis_name="s",
                                  num_cores=NC, num_subcores=NS)

    @functools.partial(
        pl.kernel,
        out_type=jax.ShapeDtypeStruct((NW, G, D), jnp.float32),
        mesh=mesh,
        scratch_types=[
            pltpu.VMEM((G, D), jnp.float32),
            pltpu.VMEM((GW, D), jnp.float32),
            pltpu.VMEM((GW * GRP,), jnp.int32),
            pltpu.VMEM((GRP, D), jnp.float32),
            pltpu.VMEM((TAIL * GRP,), jnp.int32),
            pltpu.SemaphoreType.DMA,
            pltpu.SemaphoreType.DMA,
        ],
    )
    def seg_max(xg_hbm, x_hbm, b_hbm, out_hbm,
                acc_v, xg_v, ids_v, raw_v, tids_v, sem_g, sem_i):
        wid = lax.axis_index("c") * NS + lax.axis_index("s")
        NV = D // L

        gbase = ((wid * (NGA - GW)) // (NW - 1)) // 8 * 8

        neg_inf = jnp.full((L,), -jnp.inf, dtype=jnp.float32)

        @pl.loop(0, G)
        def _init(g):
            for j in range(NV):
                acc_v[g, pl.ds(j * L, L)] = neg_inf

        def rmw(row, vals):
            for j in range(NV):
                sl = pl.ds(j * L, L)
                acc_v[row, sl] = jnp.maximum(acc_v[row, sl], vals[j])

        pltpu.async_copy(xg_hbm.at[pl.ds(gbase, GW)], xg_v, sem_g)
        pltpu.async_copy(b_hbm.at[pl.ds(gbase * GRP, GW * GRP)], ids_v,
                         sem_i)
        pltpu.make_async_copy(xg_hbm.at[pl.ds(gbase, GW)], xg_v,
                              sem_g).wait()
        pltpu.make_async_copy(b_hbm.at[pl.ds(gbase * GRP, GW * GRP)],
                              ids_v, sem_i).wait()

        @pl.loop(0, GW)
        def _grp(gi):
            idv = ids_v[pl.ds(gi * GRP, GRP)]
            first = idv[0]
            last = idv[GRP - 1]

            @pl.when(first == last)
            def _fast():
                rmw(first, tuple(xg_v[gi, pl.ds(j * L, L)]
                                 for j in range(NV)))

            @pl.when(first != last)
            def _slow():
                pltpu.sync_copy(
                    x_hbm.at[pl.ds((gbase + gi) * GRP, GRP)], raw_v)
                for t in range(GRP):
                    rmw(idv[t], tuple(raw_v[t, pl.ds(j * L, L)]
                                      for j in range(NV)))

        @pl.when(wid == NW - 1)
        def _tail():
            pltpu.sync_copy(b_hbm.at[pl.ds(NGA * GRP, TAIL * GRP)],
                            tids_v)
            for g in range(TAIL):
                pltpu.sync_copy(
                    x_hbm.at[pl.ds((NGA + g) * GRP, GRP)], raw_v)
                tidv = tids_v[pl.ds(g * GRP, GRP)]
                for t in range(GRP):
                    rmw(tidv[t], tuple(raw_v[t, pl.ds(j * L, L)]
                                       for j in range(NV)))

        pltpu.sync_copy(acc_v.at[pl.ds(0, G)], out_hbm.at[wid])

    return seg_max(xg, x, batch)


def _tc_combine_matmul(partial, W):
    def body(p_ref, w_ref, out_ref):
        hg = jnp.max(p_ref[...], axis=0)
        out_ref[...] = lax.dot_general(
            hg, w_ref[...], (((1,), (1,)), ((), ())),
            preferred_element_type=jnp.float32)

    return pl.pallas_call(
        body,
        out_shape=jax.ShapeDtypeStruct((G, W.shape[0]), jnp.float32),
    )(partial, W)


def kernel(x, batch, W):
    xg = _tc_group_max(x)
    partial = _sc_segment_max_partial(xg, x, batch.astype(jnp.int32))
    logits = _tc_combine_matmul(partial, W)
    return (logits, logits)


# speedup vs baseline: 1.1284x; 1.1284x over previous
import functools

import jax
import jax.numpy as jnp
from jax import lax
from jax.experimental import pallas as pl
from jax.experimental.pallas import tpu as pltpu
from jax.experimental.pallas import tpu_sc as plsc

N = 100000
D = 128
G = 256
NC = 2
NS = 16
NW = NC * NS
L = 16

CHUNK = 320
NBUF = 2
CPW = 10
SPAN = CPW * CHUNK
GPC = CHUNK // L


def _sc_segment_max_partial(x, batch):
    mesh = plsc.VectorSubcoreMesh(core_axis_name="c", subcore_axis_name="s",
                                  num_cores=NC, num_subcores=NS)

    @functools.partial(
        pl.kernel,
        out_type=jax.ShapeDtypeStruct((NW, G, D), jnp.float32),
        mesh=mesh,
        scratch_types=[
            pltpu.VMEM((G, D), jnp.float32),
            pltpu.VMEM((NBUF, CHUNK, D), jnp.float32),
            pltpu.VMEM((SPAN,), jnp.int32),
            pltpu.SemaphoreType.DMA((NBUF,)),
            pltpu.SemaphoreType.DMA,
        ],
    )
    def seg_max(x_hbm, b_hbm, out_hbm, acc_v, xb_v, ids_v, sem_x, sem_i):
        wid = lax.axis_index("c") * NS + lax.axis_index("s")
        NV = D // L

        rbase = ((wid * (N - SPAN)) // (NW - 1)) // 8 * 8

        neg_inf = jnp.full((L,), -jnp.inf, dtype=jnp.float32)

        @pl.loop(0, G)
        def _init(g):
            for j in range(NV):
                acc_v[g, pl.ds(j * L, L)] = neg_inf

        def rmw(row, vals):
            for j in range(NV):
                sl = pl.ds(j * L, L)
                acc_v[row, sl] = jnp.maximum(acc_v[row, sl], vals[j])

        pltpu.async_copy(b_hbm.at[pl.ds(rbase, SPAN)], ids_v, sem_i)

        def start_dma(k, buf):
            base = rbase + k * CHUNK
            pltpu.async_copy(x_hbm.at[pl.ds(base, CHUNK)],
                             xb_v.at[buf], sem_x.at[buf])

        def wait_dma(k, buf):
            base = rbase + k * CHUNK
            pltpu.make_async_copy(x_hbm.at[pl.ds(base, CHUNK)],
                                  xb_v.at[buf], sem_x.at[buf]).wait()

        def process_chunk(k, buf):
            @pl.loop(0, GPC)
            def _grp(gi):
                row0 = gi * L
                idv = ids_v[pl.ds(k * CHUNK + row0, L)]
                first = idv[0]
                last = idv[L - 1]

                @pl.when(first == last)
                def _fast():
                    accs = tuple(xb_v[buf, row0, pl.ds(j * L, L)]
                                 for j in range(NV))
                    for t in range(1, L):
                        accs = tuple(
                            jnp.maximum(a, xb_v[buf, row0 + t,
                                                pl.ds(j * L, L)])
                            for j, a in enumerate(accs))
                    rmw(first, accs)

                @pl.when(first != last)
                def _slow():
                    for t in range(L):
                        bt = idv[t]
                        rmw(bt, tuple(xb_v[buf, row0 + t, pl.ds(j * L, L)]
                                      for j in range(NV)))

        for b in range(NBUF - 1):
            start_dma(b, b)

        pltpu.make_async_copy(b_hbm.at[pl.ds(rbase, SPAN)], ids_v,
                              sem_i).wait()

        @pl.loop(0, CPW // NBUF)
        def _ring(q):
            k0 = NBUF * q
            for b in range(NBUF):
                k = k0 + b

                @pl.when(k + (NBUF - 1) < CPW)
                def _():
                    start_dma(k + (NBUF - 1), (b + NBUF - 1) % NBUF)

                wait_dma(k, b)
                process_chunk(k, b)

        pltpu.sync_copy(acc_v.at[pl.ds(0, G)], out_hbm.at[wid])

    return seg_max(x, batch)


def _tc_combine_matmul(partial, W):
    def body(p_ref, w_ref, out_ref):
        hg = jnp.max(p_ref[...], axis=0)
        out_ref[...] = lax.dot_general(
            hg, w_ref[...], (((1,), (1,)), ((), ())),
            preferred_element_type=jnp.float32)

    return pl.pallas_call(
        body,
        out_shape=jax.ShapeDtypeStruct((G, W.shape[0]), jnp.float32),
    )(partial, W)


def kernel(x, batch, W):
    partial = _sc_segment_max_partial(x, batch.astype(jnp.int32))
    logits = _tc_combine_matmul(partial, W)
    return (logits, logits)
